# Initial kernel scaffold; baseline (speedup 1.0000x reference)
import jax
import jax.numpy as jnp
from jax.experimental import pallas as pl


def kernel(pos, x, batch, W_head, b_head, W_b1, b_b1, W_b2, b_b2, W_fus, b_fus, W_p1, b_p1, W_p2, b_p2, W_p3, b_p3):
    raise NotImplementedError("write your pallas kernel here")


# speedup vs baseline: 5.0497x; 5.0497x over previous
import functools

import jax
import jax.numpy as jnp
import numpy as np
from jax import lax
from jax.experimental import pallas as pl
from jax.experimental.pallas import tpu as pltpu
from jax.experimental.pallas import tpu_sc as plsc

N = 10000
NPAD = 10240
KNN = 16
CH = 64
ROWS = 64
_HI = np.float32(np.inf)
_BIGI = np.int32(2 ** 30)


def _dot(a, b):
  return lax.dot_general(a, b, (((1,), (0,)), ((), ())),
                         preferred_element_type=jnp.float32)


def _knn_body(k, f_ref, ft_ref, idx_ref, dist_ref):
  row0 = pl.program_id(0) * ROWS
  fr = f_ref[...]
  ft = ft_ref[...]
  sq = jnp.sum(ft * ft, axis=0, keepdims=True)
  frt = lax.transpose(fr, (1, 0))
  sq_i = lax.transpose(
      jnp.sum(frt * frt, axis=0, keepdims=True), (1, 0))
  CC = 2048
  for c0 in range(0, NPAD, CC):
    d = _dot(fr, ft[:, c0:c0 + CC])
    col = lax.broadcasted_iota(jnp.int32, (ROWS, CC), 1) + c0
    rid = row0 + lax.broadcasted_iota(jnp.int32, (ROWS, CC), 0)
    dd = (sq_i - 2.0 * d) + sq[:, c0:c0 + CC]
    dd = jnp.where((col == rid) | (col >= N), _HI, dd)
    dist_ref[:, c0:c0 + CC] = dd
  dmat = dist_ref[...]
  col_all = lax.broadcasted_iota(jnp.int32, (ROWS, NPAD), 1)
  outs = []
  for _ in range(k):
    m = jnp.min(dmat, axis=1, keepdims=True)
    eq = dmat == m
    j = jnp.min(jnp.where(eq, col_all, _BIGI), axis=1, keepdims=True)
    outs.append(j)
    dmat = jnp.where(col_all == j, _HI, dmat)
  idx_ref[...] = jnp.concatenate(outs, axis=1)


def _knn(f, ft, k):
  c_dim = f.shape[1]
  return pl.pallas_call(
      functools.partial(_knn_body, k),
      grid=(NPAD // ROWS,),
      in_specs=[
          pl.BlockSpec((ROWS, c_dim), lambda i: (i, 0)),
          pl.BlockSpec((c_dim, NPAD), lambda i: (0, 0)),
      ],
      out_specs=pl.BlockSpec((ROWS, k), lambda i: (i, 0)),
      out_shape=jax.ShapeDtypeStruct((NPAD, k), jnp.int32),
      scratch_shapes=[pltpu.VMEM((ROWS, NPAD), jnp.float32)],
  )(f, ft)


_NW = 32
_PERW = NPAD // _NW
_GC = 64


def _gather_ranks(table, idxt_flat):
  mesh = plsc.VectorSubcoreMesh(core_axis_name="c", subcore_axis_name="s")

  @functools.partial(
      pl.kernel, mesh=mesh,
      out_type=jax.ShapeDtypeStruct((KNN * NPAD, 128), jnp.float32),
      scratch_types=[
          pltpu.VMEM((_GC,), jnp.int32),
          pltpu.VMEM((_GC, 128), jnp.float32),
          pltpu.SemaphoreType.DMA,
      ],
  )
  def gather_kernel(tab_h, idx_h, out_h, idx_v, rows_v, sem):
    wid = lax.axis_index("s") * 2 + lax.axis_index("c")
    base = wid * _PERW

    for r in range(KNN):
      def chunk_body(ch, carry, r=r):
        nb = base + ch * _GC
        pltpu.sync_copy(idx_h.at[pl.ds(r * NPAD + nb, _GC)], idx_v)
        pltpu.async_copy(tab_h.at[idx_v], rows_v, sem).wait()
        pltpu.sync_copy(rows_v, out_h.at[pl.ds(r * NPAD + nb, _GC)])
        return carry

      lax.fori_loop(0, _PERW // _GC, chunk_body, 0)

  return gather_kernel(table, idxt_flat)


_ET = 256


def _edge_body(c_pad, has_res, f_ref, xj_ref, w_ref, b_ref, *rest):
  if has_res:
    fp_ref, fout_ref, ftab_ref = rest
  else:
    fout_ref, ftab_ref = rest
  fb = f_ref[...]
  m = None
  for r in range(KNN):
    xj = xj_ref[r]
    e = jnp.concatenate([fb, xj[:, :c_pad] - fb], axis=1)
    h = jnp.maximum(_dot(e, w_ref[...]) + b_ref[...], 0.0)
    m = h if m is None else jnp.maximum(m, h)
  if has_res:
    m = m + fp_ref[...]
  fout_ref[...] = m
  ftab_ref[...] = jnp.concatenate(
      [m, jnp.zeros((_ET, 128 - CH), jnp.float32)], axis=1)


def _edge(f, xj, w, b, f_prev):
  c_pad = f.shape[1]
  has_res = f_prev is not None
  args = [f, xj, w, b] + ([f_prev] if has_res else [])
  in_specs = [
      pl.BlockSpec((_ET, c_pad), lambda i: (i, 0)),
      pl.BlockSpec((KNN, _ET, 128), lambda i: (0, i, 0)),
      pl.BlockSpec((2 * c_pad, CH), lambda i: (0, 0)),
      pl.BlockSpec((1, CH), lambda i: (0, 0)),
  ] + ([pl.BlockSpec((_ET, CH), lambda i: (i, 0))] if has_res else [])
  return pl.pallas_call(
      functools.partial(_edge_body, c_pad, has_res),
      grid=(NPAD // _ET,),
      in_specs=in_specs,
      out_specs=[
          pl.BlockSpec((_ET, CH), lambda i: (i, 0)),
          pl.BlockSpec((_ET, 128), lambda i: (i, 0)),
      ],
      out_shape=[
          jax.ShapeDtypeStruct((NPAD, CH), jnp.float32),
          jax.ShapeDtypeStruct((NPAD, 128), jnp.float32),
      ],
  )(*args)


def _tail_body(f1_ref, f2_ref, f3_ref,
               wf1_ref, wf2_ref, wf3_ref, bf_ref,
               wa1_ref, wa2_ref, wa3_ref, wb_ref, b1_ref,
               w2_ref, b2_ref, w3_ref, b3_ref, out_ref):
  f1 = f1_ref[...]
  f2 = f2_ref[...]
  f3 = f3_ref[...]
  s = (_dot(f1, wf1_ref[...]) + _dot(f2, wf2_ref[...]) +
       _dot(f3, wf3_ref[...]) + bf_ref[...])
  fusion = jnp.max(jnp.maximum(s, 0.0), axis=1, keepdims=True)
  h = (_dot(f1, wa1_ref[...]) + _dot(f2, wa2_ref[...]) +
       _dot(f3, wa3_ref[...]) + fusion * wb_ref[...] + b1_ref[...])
  h = jnp.maximum(h, 0.0)
  h = jnp.maximum(_dot(h, w2_ref[...]) + b2_ref[...], 0.0)
  out_ref[...] = _dot(h, w3_ref[...]) + b3_ref[...]


def _tail(f1, f2, f3, wf_parts, bf, wa_parts, wb, b1, w2, b2, w3p, b3p):
  t = 512
  wf1, wf2, wf3 = wf_parts
  wa1, wa2, wa3 = wa_parts
  row = lambda i: (i, 0)
  fix = lambda i: (0, 0)
  return pl.pallas_call(
      _tail_body,
      grid=(NPAD // t,),
      in_specs=[
          pl.BlockSpec((t, CH), row), pl.BlockSpec((t, CH), row),
          pl.BlockSpec((t, CH), row),
          pl.BlockSpec((CH, 1024), fix), pl.BlockSpec((CH, 1024), fix),
          pl.BlockSpec((CH, 1024), fix), pl.BlockSpec((1, 1024), fix),
          pl.BlockSpec((CH, 512), fix), pl.BlockSpec((CH, 512), fix),
          pl.BlockSpec((CH, 512), fix), pl.BlockSpec((1, 512), fix),
          pl.BlockSpec((1, 512), fix),
          pl.BlockSpec((512, 256), fix), pl.BlockSpec((1, 256), fix),
          pl.BlockSpec((256, 128), fix), pl.BlockSpec((1, 128), fix),
      ],
      out_specs=pl.BlockSpec((t, 128), row),
      out_shape=jax.ShapeDtypeStruct((NPAD, 128), jnp.float32),
  )(f1, f2, f3, wf1, wf2, wf3, bf, wa1, wa2, wa3, wb, b1, w2, b2, w3p, b3p)


def kernel(pos, x, batch, W_head, b_head, W_b1, b_b1, W_b2, b_b2,
           W_fus, b_fus, W_p1, b_p1, W_p2, b_p2, W_p3, b_p3):
  x0 = jnp.concatenate([pos, x], axis=1)
  x0p = jnp.zeros((NPAD, 16), jnp.float32).at[:N, :9].set(x0)
  x0tab = jnp.zeros((NPAD, 128), jnp.float32).at[:N, :9].set(x0)
  posp = jnp.zeros((NPAD, 8), jnp.float32).at[:N, :3].set(pos)

  w32h = (jnp.zeros((32, CH), jnp.float32)
          .at[:9].set(W_head[:9]).at[16:25].set(W_head[9:]))
  wf_parts = (W_fus[:CH], W_fus[CH:2 * CH], W_fus[2 * CH:])
  wa_parts = (W_p1[:CH], W_p1[CH:2 * CH], W_p1[2 * CH:3 * CH])
  wb = W_p1[3 * CH:3 * CH + 1]
  w3p = jnp.zeros((256, 128), jnp.float32).at[:, :13].set(W_p3)
  b3p = jnp.zeros((1, 128), jnp.float32).at[0, :13].set(b_p3)
  b2d = lambda v: v.reshape(1, -1)

  idx0 = _knn(posp, posp.T, KNN)
  xj0 = _gather_ranks(x0tab, idx0.T.reshape(-1))
  f1, f1tab = _edge(x0p, xj0.reshape(KNN, NPAD, 128), w32h, b2d(b_head), None)

  idx1 = _knn(f1, f1.T, KNN)
  xj1 = _gather_ranks(f1tab, idx1.T.reshape(-1))
  f2, f2tab = _edge(f1, xj1.reshape(KNN, NPAD, 128), W_b1, b2d(b_b1), f1)

  idx2 = _knn(f2, f2.T, 2 * KNN)[:, ::2]
  xj2 = _gather_ranks(f2tab, idx2.T.reshape(-1))
  f3, _ = _edge(f2, xj2.reshape(KNN, NPAD, 128), W_b2, b2d(b_b2), f2)

  out = _tail(f1, f2, f3, wf_parts, b2d(b_fus), wa_parts, wb,
              b2d(b_p1), W_p2, b2d(b_p2), w3p, b3p)
  return out[:N, :13]


# speedup vs baseline: 5.0919x; 1.0083x over previous
import functools

import jax
import jax.numpy as jnp
import numpy as np
from jax import lax
from jax.experimental import pallas as pl
from jax.experimental.pallas import tpu as pltpu
from jax.experimental.pallas import tpu_sc as plsc

N = 10000
NPAD = 10240
KNN = 16
CH = 64
ROWS = 128
_HI = np.float32(np.inf)
_BIGI = np.int32(2 ** 30)


def _dot(a, b):
  return lax.dot_general(a, b, (((1,), (0,)), ((), ())),
                         preferred_element_type=jnp.float32)


def _knn_body(k, f_ref, ft_ref, idx_ref, dist_ref):
  row0 = pl.program_id(0) * ROWS
  fr = f_ref[...]
  ft = ft_ref[...]
  sq = jnp.sum(ft * ft, axis=0, keepdims=True)
  frt = lax.transpose(fr, (1, 0))
  sq_i = lax.transpose(
      jnp.sum(frt * frt, axis=0, keepdims=True), (1, 0))
  CC = 2048
  for c0 in range(0, NPAD, CC):
    d = _dot(fr, ft[:, c0:c0 + CC])
    col = lax.broadcasted_iota(jnp.int32, (ROWS, CC), 1) + c0
    rid = row0 + lax.broadcasted_iota(jnp.int32, (ROWS, CC), 0)
    dd = (sq_i - 2.0 * d) + sq[:, c0:c0 + CC]
    dd = jnp.where((col == rid) | (col >= N), _HI, dd)
    dist_ref[:, c0:c0 + CC] = dd
  dmat = dist_ref[...]
  col_all = lax.broadcasted_iota(jnp.int32, (ROWS, NPAD), 1)
  outs = []
  for _ in range(k):
    m = jnp.min(dmat, axis=1, keepdims=True)
    eq = dmat == m
    j = jnp.min(jnp.where(eq, col_all, _BIGI), axis=1, keepdims=True)
    outs.append(j)
    dmat = jnp.where(col_all == j, _HI, dmat)
  idx_ref[...] = jnp.concatenate(outs, axis=1)


def _knn(f, ft, k):
  c_dim = f.shape[1]
  return pl.pallas_call(
      functools.partial(_knn_body, k),
      grid=(NPAD // ROWS,),
      in_specs=[
          pl.BlockSpec((ROWS, c_dim), lambda i: (i, 0)),
          pl.BlockSpec((c_dim, NPAD), lambda i: (0, 0)),
      ],
      out_specs=pl.BlockSpec((ROWS, k), lambda i: (i, 0)),
      out_shape=jax.ShapeDtypeStruct((NPAD, k), jnp.int32),
      scratch_shapes=[pltpu.VMEM((ROWS, NPAD), jnp.float32)],
  )(f, ft)


_NW = 32
_PERW = NPAD // _NW
_GC = 64


def _gather_ranks(table, idxt_flat):
  mesh = plsc.VectorSubcoreMesh(core_axis_name="c", subcore_axis_name="s")

  @functools.partial(
      pl.kernel, mesh=mesh,
      out_type=jax.ShapeDtypeStruct((KNN * NPAD, 128), jnp.float32),
      scratch_types=[
          pltpu.VMEM((2, _GC), jnp.int32),
          pltpu.VMEM((2, _GC, 128), jnp.float32),
          pltpu.SemaphoreType.DMA((2,)),
          pltpu.SemaphoreType.DMA((2,)),
      ],
  )
  def gather_kernel(tab_h, idx_h, out_h, idx_v, rows_v, gsem, osem):
    wid = lax.axis_index("s") * 2 + lax.axis_index("c")
    base = wid * _PERW
    nch = _PERW // _GC
    ng = KNN * nch
    offs = [(g // nch) * NPAD + base + (g % nch) * _GC for g in range(ng)]
    gather = [None] * ng
    outcp = [None] * ng

    def issue(g):
      p = g % 2
      if g >= 2:
        outcp[g - 2].wait()
      pltpu.sync_copy(idx_h.at[pl.ds(offs[g], _GC)], idx_v.at[p])
      gather[g] = pltpu.async_copy(tab_h.at[idx_v.at[p]], rows_v.at[p],
                                   gsem.at[p])

    issue(0)
    for g in range(ng):
      if g + 1 < ng:
        issue(g + 1)
      p = g % 2
      gather[g].wait()
      outcp[g] = pltpu.async_copy(rows_v.at[p], out_h.at[pl.ds(offs[g], _GC)],
                                  osem.at[p])
    outcp[ng - 2].wait()
    outcp[ng - 1].wait()

  return gather_kernel(table, idxt_flat)


_ET = 256


def _edge_body(c_pad, has_res, f_ref, xj_ref, w_ref, b_ref, *rest):
  if has_res:
    fp_ref, fout_ref, ftab_ref = rest
  else:
    fout_ref, ftab_ref = rest
  fb = f_ref[...]
  m = None
  for r in range(KNN):
    xj = xj_ref[r]
    e = jnp.concatenate([fb, xj[:, :c_pad] - fb], axis=1)
    h = jnp.maximum(_dot(e, w_ref[...]) + b_ref[...], 0.0)
    m = h if m is None else jnp.maximum(m, h)
  if has_res:
    m = m + fp_ref[...]
  fout_ref[...] = m
  ftab_ref[...] = jnp.concatenate(
      [m, jnp.zeros((_ET, 128 - CH), jnp.float32)], axis=1)


def _edge(f, xj, w, b, f_prev):
  c_pad = f.shape[1]
  has_res = f_prev is not None
  args = [f, xj, w, b] + ([f_prev] if has_res else [])
  in_specs = [
      pl.BlockSpec((_ET, c_pad), lambda i: (i, 0)),
      pl.BlockSpec((KNN, _ET, 128), lambda i: (0, i, 0)),
      pl.BlockSpec((2 * c_pad, CH), lambda i: (0, 0)),
      pl.BlockSpec((1, CH), lambda i: (0, 0)),
  ] + ([pl.BlockSpec((_ET, CH), lambda i: (i, 0))] if has_res else [])
  return pl.pallas_call(
      functools.partial(_edge_body, c_pad, has_res),
      grid=(NPAD // _ET,),
      in_specs=in_specs,
      out_specs=[
          pl.BlockSpec((_ET, CH), lambda i: (i, 0)),
          pl.BlockSpec((_ET, 128), lambda i: (i, 0)),
      ],
      out_shape=[
          jax.ShapeDtypeStruct((NPAD, CH), jnp.float32),
          jax.ShapeDtypeStruct((NPAD, 128), jnp.float32),
      ],
  )(*args)


def _tail_body(f1_ref, f2_ref, f3_ref,
               wf1_ref, wf2_ref, wf3_ref, bf_ref,
               wa1_ref, wa2_ref, wa3_ref, wb_ref, b1_ref,
               w2_ref, b2_ref, w3_ref, b3_ref, out_ref):
  f1 = f1_ref[...]
  f2 = f2_ref[...]
  f3 = f3_ref[...]
  s = (_dot(f1, wf1_ref[...]) + _dot(f2, wf2_ref[...]) +
       _dot(f3, wf3_ref[...]) + bf_ref[...])
  fusion = jnp.max(jnp.maximum(s, 0.0), axis=1, keepdims=True)
  h = (_dot(f1, wa1_ref[...]) + _dot(f2, wa2_ref[...]) +
       _dot(f3, wa3_ref[...]) + fusion * wb_ref[...] + b1_ref[...])
  h = jnp.maximum(h, 0.0)
  h = jnp.maximum(_dot(h, w2_ref[...]) + b2_ref[...], 0.0)
  out_ref[...] = _dot(h, w3_ref[...]) + b3_ref[...]


def _tail(f1, f2, f3, wf_parts, bf, wa_parts, wb, b1, w2, b2, w3p, b3p):
  t = 512
  wf1, wf2, wf3 = wf_parts
  wa1, wa2, wa3 = wa_parts
  row = lambda i: (i, 0)
  fix = lambda i: (0, 0)
  return pl.pallas_call(
      _tail_body,
      grid=(NPAD // t,),
      in_specs=[
          pl.BlockSpec((t, CH), row), pl.BlockSpec((t, CH), row),
          pl.BlockSpec((t, CH), row),
          pl.BlockSpec((CH, 1024), fix), pl.BlockSpec((CH, 1024), fix),
          pl.BlockSpec((CH, 1024), fix), pl.BlockSpec((1, 1024), fix),
          pl.BlockSpec((CH, 512), fix), pl.BlockSpec((CH, 512), fix),
          pl.BlockSpec((CH, 512), fix), pl.BlockSpec((1, 512), fix),
          pl.BlockSpec((1, 512), fix),
          pl.BlockSpec((512, 256), fix), pl.BlockSpec((1, 256), fix),
          pl.BlockSpec((256, 128), fix), pl.BlockSpec((1, 128), fix),
      ],
      out_specs=pl.BlockSpec((t, 128), row),
      out_shape=jax.ShapeDtypeStruct((NPAD, 128), jnp.float32),
  )(f1, f2, f3, wf1, wf2, wf3, bf, wa1, wa2, wa3, wb, b1, w2, b2, w3p, b3p)


def kernel(pos, x, batch, W_head, b_head, W_b1, b_b1, W_b2, b_b2,
           W_fus, b_fus, W_p1, b_p1, W_p2, b_p2, W_p3, b_p3):
  x0 = jnp.concatenate([pos, x], axis=1)
  x0p = jnp.zeros((NPAD, 16), jnp.float32).at[:N, :9].set(x0)
  x0tab = jnp.zeros((NPAD, 128), jnp.float32).at[:N, :9].set(x0)
  posp = jnp.zeros((NPAD, 8), jnp.float32).at[:N, :3].set(pos)

  w32h = (jnp.zeros((32, CH), jnp.float32)
          .at[:9].set(W_head[:9]).at[16:25].set(W_head[9:]))
  wf_parts = (W_fus[:CH], W_fus[CH:2 * CH], W_fus[2 * CH:])
  wa_parts = (W_p1[:CH], W_p1[CH:2 * CH], W_p1[2 * CH:3 * CH])
  wb = W_p1[3 * CH:3 * CH + 1]
  w3p = jnp.zeros((256, 128), jnp.float32).at[:, :13].set(W_p3)
  b3p = jnp.zeros((1, 128), jnp.float32).at[0, :13].set(b_p3)
  b2d = lambda v: v.reshape(1, -1)

  idx0 = _knn(posp, posp.T, KNN)
  xj0 = _gather_ranks(x0tab, idx0.T.reshape(-1))
  f1, f1tab = _edge(x0p, xj0.reshape(KNN, NPAD, 128), w32h, b2d(b_head), None)

  idx1 = _knn(f1, f1.T, KNN)
  xj1 = _gather_ranks(f1tab, idx1.T.reshape(-1))
  f2, f2tab = _edge(f1, xj1.reshape(KNN, NPAD, 128), W_b1, b2d(b_b1), f1)

  idx2 = _knn(f2, f2.T, 2 * KNN)[:, ::2]
  xj2 = _gather_ranks(f2tab, idx2.T.reshape(-1))
  f3, _ = _edge(f2, xj2.reshape(KNN, NPAD, 128), W_b2, b2d(b_b2), f2)

  out = _tail(f1, f2, f3, wf_parts, b2d(b_fus), wa_parts, wb,
              b2d(b_p1), W_p2, b2d(b_p2), w3p, b3p)
  return out[:N, :13]


# speedup vs baseline: 5.4639x; 1.0731x over previous
import functools

import jax
import jax.numpy as jnp
import numpy as np
from jax import lax
from jax.experimental import pallas as pl
from jax.experimental.pallas import tpu as pltpu
from jax.experimental.pallas import tpu_sc as plsc

N = 10000
NPAD = 10240
KNN = 16
CH = 64
ROWS = 128
_HI = np.float32(np.inf)
_BIGI = np.int32(2 ** 30)


def _dot(a, b):
  return lax.dot_general(a, b, (((1,), (0,)), ((), ())),
                         preferred_element_type=jnp.float32)


def _knn_body(k, f_ref, ft_ref, idx_ref, dist_ref):
  row0 = pl.program_id(0) * ROWS
  fr = f_ref[...]
  ft = ft_ref[...]
  sq = jnp.sum(ft * ft, axis=0, keepdims=True)
  frt = lax.transpose(fr, (1, 0))
  sq_i = lax.transpose(
      jnp.sum(frt * frt, axis=0, keepdims=True), (1, 0))
  CC = 2048
  for c0 in range(0, NPAD, CC):
    d = _dot(fr, ft[:, c0:c0 + CC])
    col = lax.broadcasted_iota(jnp.int32, (ROWS, CC), 1) + c0
    rid = row0 + lax.broadcasted_iota(jnp.int32, (ROWS, CC), 0)
    dd = (sq_i - 2.0 * d) + sq[:, c0:c0 + CC]
    dd = jnp.where((col == rid) | (col >= N), _HI, dd)
    dist_ref[:, c0:c0 + CC] = dd
  dmat = dist_ref[...]
  col_all = lax.broadcasted_iota(jnp.int32, (ROWS, NPAD), 1)
  outs = []
  for _ in range(k):
    m = jnp.min(dmat, axis=1, keepdims=True)
    eq = dmat == m
    j = jnp.min(jnp.where(eq, col_all, _BIGI), axis=1, keepdims=True)
    outs.append(j)
    dmat = jnp.where(col_all == j, _HI, dmat)
  idx_ref[...] = jnp.concatenate(outs, axis=1)


def _knn(f, ft, k):
  c_dim = f.shape[1]
  return pl.pallas_call(
      functools.partial(_knn_body, k),
      grid=(NPAD // ROWS,),
      in_specs=[
          pl.BlockSpec((ROWS, c_dim), lambda i: (i, 0)),
          pl.BlockSpec((c_dim, NPAD), lambda i: (0, 0)),
      ],
      out_specs=pl.BlockSpec((ROWS, k), lambda i: (i, 0)),
      out_shape=jax.ShapeDtypeStruct((NPAD, k), jnp.int32),
      scratch_shapes=[pltpu.VMEM((ROWS, NPAD), jnp.float32)],
  )(f, ft)


_NW = 32
_WROWS = KNN * NPAD // _NW
_GC = 128
_NBUF = 3


def _gather_ranks(table, idxt_flat):
  mesh = plsc.VectorSubcoreMesh(core_axis_name="c", subcore_axis_name="s")

  @functools.partial(
      pl.kernel, mesh=mesh,
      out_type=jax.ShapeDtypeStruct((KNN * NPAD, 128), jnp.float32),
      scratch_types=[
          pltpu.VMEM((_NBUF, _GC), jnp.int32),
          pltpu.VMEM((_NBUF, _GC, 128), jnp.float32),
          pltpu.SemaphoreType.DMA((_NBUF,)),
          pltpu.SemaphoreType.DMA((_NBUF,)),
      ],
  )
  def gather_kernel(tab_h, idx_h, out_h, idx_v, rows_v, gsem, osem):
    wid = lax.axis_index("s") * 2 + lax.axis_index("c")
    base = wid * _WROWS
    ng = _WROWS // _GC
    gather = [None] * ng
    outcp = [None] * ng

    def issue(g):
      p = g % _NBUF
      if g >= _NBUF:
        outcp[g - _NBUF].wait()
      pltpu.sync_copy(idx_h.at[pl.ds(base + g * _GC, _GC)], idx_v.at[p])
      gather[g] = pltpu.async_copy(tab_h.at[idx_v.at[p]], rows_v.at[p],
                                   gsem.at[p])

    for g in range(_NBUF - 1):
      issue(g)
    for g in range(ng):
      if g + _NBUF - 1 < ng:
        issue(g + _NBUF - 1)
      p = g % _NBUF
      gather[g].wait()
      outcp[g] = pltpu.async_copy(
          rows_v.at[p], out_h.at[pl.ds(base + g * _GC, _GC)], osem.at[p])
    for g in range(ng - _NBUF, ng):
      outcp[g].wait()

  return gather_kernel(table, idxt_flat)


_ET = 256


def _edge_body(c_pad, has_res, f_ref, xj_ref, w_ref, b_ref, *rest):
  if has_res:
    fp_ref, fout_ref, ftab_ref = rest
  else:
    fout_ref, ftab_ref = rest
  fb = f_ref[...]
  m = None
  for r in range(KNN):
    xj = xj_ref[r]
    e = jnp.concatenate([fb, xj[:, :c_pad] - fb], axis=1)
    h = jnp.maximum(_dot(e, w_ref[...]) + b_ref[...], 0.0)
    m = h if m is None else jnp.maximum(m, h)
  if has_res:
    m = m + fp_ref[...]
  fout_ref[...] = m
  ftab_ref[...] = jnp.concatenate(
      [m, jnp.zeros((_ET, 128 - CH), jnp.float32)], axis=1)


def _edge(f, xj, w, b, f_prev):
  c_pad = f.shape[1]
  has_res = f_prev is not None
  args = [f, xj, w, b] + ([f_prev] if has_res else [])
  in_specs = [
      pl.BlockSpec((_ET, c_pad), lambda i: (i, 0)),
      pl.BlockSpec((KNN, _ET, 128), lambda i: (0, i, 0)),
      pl.BlockSpec((2 * c_pad, CH), lambda i: (0, 0)),
      pl.BlockSpec((1, CH), lambda i: (0, 0)),
  ] + ([pl.BlockSpec((_ET, CH), lambda i: (i, 0))] if has_res else [])
  return pl.pallas_call(
      functools.partial(_edge_body, c_pad, has_res),
      grid=(NPAD // _ET,),
      in_specs=in_specs,
      out_specs=[
          pl.BlockSpec((_ET, CH), lambda i: (i, 0)),
          pl.BlockSpec((_ET, 128), lambda i: (i, 0)),
      ],
      out_shape=[
          jax.ShapeDtypeStruct((NPAD, CH), jnp.float32),
          jax.ShapeDtypeStruct((NPAD, 128), jnp.float32),
      ],
  )(*args)


def _tail_body(f1_ref, f2_ref, f3_ref,
               wf1_ref, wf2_ref, wf3_ref, bf_ref,
               wa1_ref, wa2_ref, wa3_ref, wb_ref, b1_ref,
               w2_ref, b2_ref, w3_ref, b3_ref, out_ref):
  f1 = f1_ref[...]
  f2 = f2_ref[...]
  f3 = f3_ref[...]
  s = (_dot(f1, wf1_ref[...]) + _dot(f2, wf2_ref[...]) +
       _dot(f3, wf3_ref[...]) + bf_ref[...])
  fusion = jnp.max(jnp.maximum(s, 0.0), axis=1, keepdims=True)
  h = (_dot(f1, wa1_ref[...]) + _dot(f2, wa2_ref[...]) +
       _dot(f3, wa3_ref[...]) + fusion * wb_ref[...] + b1_ref[...])
  h = jnp.maximum(h, 0.0)
  h = jnp.maximum(_dot(h, w2_ref[...]) + b2_ref[...], 0.0)
  out_ref[...] = _dot(h, w3_ref[...]) + b3_ref[...]


def _tail(f1, f2, f3, wf_parts, bf, wa_parts, wb, b1, w2, b2, w3p, b3p):
  t = 512
  wf1, wf2, wf3 = wf_parts
  wa1, wa2, wa3 = wa_parts
  row = lambda i: (i, 0)
  fix = lambda i: (0, 0)
  return pl.pallas_call(
      _tail_body,
      grid=(NPAD // t,),
      in_specs=[
          pl.BlockSpec((t, CH), row), pl.BlockSpec((t, CH), row),
          pl.BlockSpec((t, CH), row),
          pl.BlockSpec((CH, 1024), fix), pl.BlockSpec((CH, 1024), fix),
          pl.BlockSpec((CH, 1024), fix), pl.BlockSpec((1, 1024), fix),
          pl.BlockSpec((CH, 512), fix), pl.BlockSpec((CH, 512), fix),
          pl.BlockSpec((CH, 512), fix), pl.BlockSpec((1, 512), fix),
          pl.BlockSpec((1, 512), fix),
          pl.BlockSpec((512, 256), fix), pl.BlockSpec((1, 256), fix),
          pl.BlockSpec((256, 128), fix), pl.BlockSpec((1, 128), fix),
      ],
      out_specs=pl.BlockSpec((t, 128), row),
      out_shape=jax.ShapeDtypeStruct((NPAD, 128), jnp.float32),
  )(f1, f2, f3, wf1, wf2, wf3, bf, wa1, wa2, wa3, wb, b1, w2, b2, w3p, b3p)


def kernel(pos, x, batch, W_head, b_head, W_b1, b_b1, W_b2, b_b2,
           W_fus, b_fus, W_p1, b_p1, W_p2, b_p2, W_p3, b_p3):
  x0 = jnp.concatenate([pos, x], axis=1)
  x0p = jnp.zeros((NPAD, 16), jnp.float32).at[:N, :9].set(x0)
  x0tab = jnp.zeros((NPAD, 128), jnp.float32).at[:N, :9].set(x0)
  posp = jnp.zeros((NPAD, 8), jnp.float32).at[:N, :3].set(pos)

  w32h = (jnp.zeros((32, CH), jnp.float32)
          .at[:9].set(W_head[:9]).at[16:25].set(W_head[9:]))
  wf_parts = (W_fus[:CH], W_fus[CH:2 * CH], W_fus[2 * CH:])
  wa_parts = (W_p1[:CH], W_p1[CH:2 * CH], W_p1[2 * CH:3 * CH])
  wb = W_p1[3 * CH:3 * CH + 1]
  w3p = jnp.zeros((256, 128), jnp.float32).at[:, :13].set(W_p3)
  b3p = jnp.zeros((1, 128), jnp.float32).at[0, :13].set(b_p3)
  b2d = lambda v: v.reshape(1, -1)

  idx0 = _knn(posp, posp.T, KNN)
  xj0 = _gather_ranks(x0tab, idx0.T.reshape(-1))
  f1, f1tab = _edge(x0p, xj0.reshape(KNN, NPAD, 128), w32h, b2d(b_head), None)

  idx1 = _knn(f1, f1.T, KNN)
  xj1 = _gather_ranks(f1tab, idx1.T.reshape(-1))
  f2, f2tab = _edge(f1, xj1.reshape(KNN, NPAD, 128), W_b1, b2d(b_b1), f1)

  idx2 = _knn(f2, f2.T, 2 * KNN)[:, ::2]
  xj2 = _gather_ranks(f2tab, idx2.T.reshape(-1))
  f3, _ = _edge(f2, xj2.reshape(KNN, NPAD, 128), W_b2, b2d(b_b2), f2)

  out = _tail(f1, f2, f3, wf_parts, b2d(b_fus), wa_parts, wb,
              b2d(b_p1), W_p2, b2d(b_p2), w3p, b3p)
  return out[:N, :13]
